# Initial kernel scaffold; baseline (speedup 1.0000x reference)
import jax
import jax.numpy as jnp
from jax.experimental import pallas as pl


def kernel(x, edge_index, edge_attr, enc1_W, enc1_b, mu_W, mu_b, lv_W, lv_b, dec1_W, dec1_b, dec2_W, dec2_b):
    raise NotImplementedError("write your pallas kernel here")


# speedup vs baseline: 8.0338x; 8.0338x over previous
import functools

import jax
import jax.numpy as jnp
from jax import lax
from jax.experimental import pallas as pl
from jax.experimental.pallas import tpu as pltpu
from jax.experimental.pallas import tpu_sc as plsc

N_NODES = 10000
N_EDGES = 320000
NC, NS, LANES = 2, 16, 16
NW = NC * NS
EPW = N_EDGES // NW
EC = 80
NCH = EPW // EC
RPT = N_NODES // NS


def _worker_ids():
    c = lax.axis_index("c")
    s = lax.axis_index("s")
    return c, s, s * NC + c


def _deg_body(dst_hbm, ew_hbm, out_hbm, dstb, ewb, zb, acc):
    c, s, w = _worker_ids()

    def zrow(i, _):
        zb[pl.ds(i * 16, 16)] = jnp.zeros((16,), jnp.float32)
        return 0
    lax.fori_loop(0, 40, zrow, 0)

    @pl.when(s < NS - 1)
    def _():
        pltpu.sync_copy(zb.at[pl.ds(0, 624)],
                        acc.at[pl.ds(pl.multiple_of(s * 624, 8), 624)])

    @pl.when(s == NS - 1)
    def _():
        pltpu.sync_copy(zb, acc.at[pl.ds((NS - 1) * 624, 640)])

    plsc.subcore_barrier()

    def chunk(i, _):
        off = pl.multiple_of(w * EPW + i * EC, 8)
        pltpu.sync_copy(dst_hbm.at[pl.ds(off, EC)], dstb)
        pltpu.sync_copy(ew_hbm.at[pl.ds(off, EC)], ewb)
        pltpu.sync_copy(ewb, acc.at[dstb], add=True)
        return 0
    lax.fori_loop(0, NCH, chunk, 0)

    plsc.subcore_barrier()

    @pl.when(s < NS - 1)
    def _():
        o = pl.multiple_of(s * 624, 8)
        pltpu.sync_copy(acc.at[pl.ds(o, 624)], zb.at[pl.ds(0, 624)])
        pltpu.sync_copy(zb.at[pl.ds(0, 624)],
                        out_hbm.at[pl.ds(c * N_NODES + o, 624)])

    @pl.when(s == NS - 1)
    def _():
        o = pl.multiple_of(c * N_NODES + (NS - 1) * 624, 8)
        pltpu.sync_copy(acc.at[pl.ds((NS - 1) * 624, 640)], zb)
        pltpu.sync_copy(zb, out_hbm.at[pl.ds(o, 640)])


_deg_call = pl.kernel(
    _deg_body,
    out_type=jax.ShapeDtypeStruct((NC * N_NODES,), jnp.float32),
    mesh=plsc.VectorSubcoreMesh(core_axis_name="c", subcore_axis_name="s"),
    compiler_params=pltpu.CompilerParams(use_tc_tiling_on_sc=False,
                                         needs_layout_passes=False),
    scratch_types=[
        pltpu.VMEM((EC,), jnp.int32),
        pltpu.VMEM((EC,), jnp.float32),
        pltpu.VMEM((640,), jnp.float32),
        pltpu.VMEM_SHARED((N_NODES,), jnp.float32),
    ],
)


NC2 = 400
NCH2 = EPW // NC2


def _norm_body(src_hbm, dst_hbm, ew_hbm, dinv_hbm, out_hbm, dv, srcb, dstb, ewb, nb):
    _, _, w = _worker_ids()
    pltpu.sync_copy(dinv_hbm, dv)

    def chunk(i, _):
        off = pl.multiple_of(w * EPW + i * NC2, 8)
        pltpu.sync_copy(src_hbm.at[pl.ds(off, NC2)], srcb)
        pltpu.sync_copy(dst_hbm.at[pl.ds(off, NC2)], dstb)
        pltpu.sync_copy(ew_hbm.at[pl.ds(off, NC2)], ewb)

        def inner(j, _):
            sl = pl.ds(j * 16, 16)
            nv = (plsc.load_gather(dv, [srcb[sl]]) * ewb[sl]
                  * plsc.load_gather(dv, [dstb[sl]]))
            nb[sl] = nv
            return 0
        lax.fori_loop(0, NC2 // 16, inner, 0)
        pltpu.sync_copy(nb, out_hbm.at[pl.ds(off, NC2)])
        return 0
    lax.fori_loop(0, NCH2, chunk, 0)


_norm_call = pl.kernel(
    _norm_body,
    out_type=jax.ShapeDtypeStruct((N_EDGES,), jnp.float32),
    mesh=plsc.VectorSubcoreMesh(core_axis_name="c", subcore_axis_name="s"),
    compiler_params=pltpu.CompilerParams(use_tc_tiling_on_sc=False,
                                         needs_layout_passes=False),
    scratch_types=[
        pltpu.VMEM((N_NODES,), jnp.float32),
        pltpu.VMEM((NC2,), jnp.int32),
        pltpu.VMEM((NC2,), jnp.int32),
        pltpu.VMEM((NC2,), jnp.float32),
        pltpu.VMEM((NC2,), jnp.float32),
    ],
)


def _spmm_body(F, x_hbm, src_hbm, dst_hbm, nrm_hbm, out_hbm,
               srcb, dstb, nb, rows, zb, sem, acc):
    c, s, w = _worker_ids()

    def zrow(i, _):
        for j in range(F // 16):
            zb[i, pl.ds(j * 16, 16)] = jnp.zeros((16,), jnp.float32)
        return 0
    lax.fori_loop(0, 104, zrow, 0)
    for k in range(6):
        pltpu.sync_copy(zb, acc.at[pl.ds(s * 624 + k * 104, 104)])

    @pl.when(s == NS - 1)
    def _():
        pltpu.sync_copy(zb.at[pl.ds(0, 16)], acc.at[pl.ds(N_NODES - 16, 16)])

    plsc.subcore_barrier()

    def chunk(i, _):
        off = pl.multiple_of(w * EPW + i * EC, 8)
        pltpu.sync_copy(src_hbm.at[pl.ds(off, EC)], srcb)
        pltpu.sync_copy(dst_hbm.at[pl.ds(off, EC)], dstb)
        pltpu.sync_copy(nrm_hbm.at[pl.ds(off, EC)], nb)
        pltpu.async_copy(x_hbm.at[srcb], rows, sem).wait()

        def scale(r, _):
            nbc = plsc.load_gather(nb, [jnp.full((LANES,), r, jnp.int32)])
            for j in range(F // 16):
                rows[r, pl.ds(j * 16, 16)] = rows[r, pl.ds(j * 16, 16)] * nbc
            return 0
        lax.fori_loop(0, EC, scale, 0)

        pltpu.sync_copy(rows, acc.at[dstb], add=True)
        return 0
    lax.fori_loop(0, NCH, chunk, 0)

    plsc.subcore_barrier()

    for k in range(6):
        o = pl.multiple_of(s * 624 + k * 104, 8)
        pltpu.sync_copy(acc.at[pl.ds(o, 104)], zb)
        pltpu.sync_copy(zb, out_hbm.at[c, pl.ds(o, 104)])

    @pl.when(s == NS - 1)
    def _():
        pltpu.sync_copy(acc.at[pl.ds(N_NODES - 16, 16)], zb.at[pl.ds(0, 16)])
        pltpu.sync_copy(zb.at[pl.ds(0, 16)],
                        out_hbm.at[c, pl.ds(N_NODES - 16, 16)])


@functools.cache
def _make_spmm(F):
    return pl.kernel(
        functools.partial(_spmm_body, F),
        out_type=jax.ShapeDtypeStruct((NC, N_NODES, F), jnp.float32),
        mesh=plsc.VectorSubcoreMesh(core_axis_name="c", subcore_axis_name="s"),
        compiler_params=pltpu.CompilerParams(use_tc_tiling_on_sc=False,
                                             needs_layout_passes=False),
        scratch_types=[
            pltpu.VMEM((EC,), jnp.int32),
            pltpu.VMEM((EC,), jnp.int32),
            pltpu.VMEM((EC,), jnp.float32),
            pltpu.VMEM((EC, F), jnp.float32),
            pltpu.VMEM((104, F), jnp.float32),
            pltpu.SemaphoreType.DMA,
            pltpu.VMEM_SHARED((N_NODES, F), jnp.float32),
        ],
    )


BS = 1000


def _dense_body(relu, y_ref, x_ref, d2_ref, w_ref, b_ref, o_ref):
    h = y_ref[0] + y_ref[1] + d2_ref[...] * x_ref[...]
    o = lax.dot_general(h, w_ref[...], (((1,), (0,)), ((), ())),
                        precision=lax.Precision.HIGHEST,
                        preferred_element_type=jnp.float32) + b_ref[...]
    o_ref[...] = jnp.maximum(o, 0.0) if relu else o


def _dense(y, x, d2, W, b, relu):
    n, fin = x.shape
    fout = W.shape[1]
    return pl.pallas_call(
        functools.partial(_dense_body, relu),
        grid=(n // BS,),
        in_specs=[
            pl.BlockSpec((NC, BS, fin), lambda i: (0, i, 0)),
            pl.BlockSpec((BS, fin), lambda i: (i, 0)),
            pl.BlockSpec((BS, 1), lambda i: (i, 0)),
            pl.BlockSpec((fin, fout), lambda i: (0, 0)),
            pl.BlockSpec((1, fout), lambda i: (0, 0)),
        ],
        out_specs=pl.BlockSpec((BS, fout), lambda i: (i, 0)),
        out_shape=jax.ShapeDtypeStruct((n, fout), jnp.float32),
    )(y, x, d2, W, b.reshape(1, -1))


def _stageb_body(y_ref, h_ref, d2_ref, wmu_ref, bmu_ref, wlv_ref, blv_ref,
                 eps_ref, mu_ref, lv_ref, z_ref):
    H = y_ref[0] + y_ref[1] + d2_ref[...] * h_ref[...]
    dn = (((1,), (0,)), ((), ()))
    mu = lax.dot_general(H, wmu_ref[...], dn, precision=lax.Precision.HIGHEST,
                         preferred_element_type=jnp.float32) + bmu_ref[...]
    lv = lax.dot_general(H, wlv_ref[...], dn, precision=lax.Precision.HIGHEST,
                         preferred_element_type=jnp.float32) + blv_ref[...]
    mu_ref[...] = mu
    lv_ref[...] = lv
    z_ref[...] = mu + eps_ref[...] * jnp.exp(0.5 * lv)


def _stageb(y, hidden, d2, mu_W, mu_b, lv_W, lv_b, eps):
    n, fin = hidden.shape
    fout = mu_W.shape[1]
    return pl.pallas_call(
        _stageb_body,
        grid=(n // BS,),
        in_specs=[
            pl.BlockSpec((NC, BS, fin), lambda i: (0, i, 0)),
            pl.BlockSpec((BS, fin), lambda i: (i, 0)),
            pl.BlockSpec((BS, 1), lambda i: (i, 0)),
            pl.BlockSpec((fin, fout), lambda i: (0, 0)),
            pl.BlockSpec((1, fout), lambda i: (0, 0)),
            pl.BlockSpec((fin, fout), lambda i: (0, 0)),
            pl.BlockSpec((1, fout), lambda i: (0, 0)),
            pl.BlockSpec((BS, fout), lambda i: (i, 0)),
        ],
        out_specs=[
            pl.BlockSpec((BS, fout), lambda i: (i, 0)),
            pl.BlockSpec((BS, fout), lambda i: (i, 0)),
            pl.BlockSpec((BS, fout), lambda i: (i, 0)),
        ],
        out_shape=[
            jax.ShapeDtypeStruct((n, fout), jnp.float32),
            jax.ShapeDtypeStruct((n, fout), jnp.float32),
            jax.ShapeDtypeStruct((n, fout), jnp.float32),
        ],
    )(y, hidden, d2, mu_W, mu_b.reshape(1, -1), lv_W, lv_b.reshape(1, -1), eps)


def kernel(x, edge_index, edge_attr, enc1_W, enc1_b, mu_W, mu_b, lv_W, lv_b,
           dec1_W, dec1_b, dec2_W, dec2_b):
    src = edge_index[0].astype(jnp.int32)
    dst = edge_index[1].astype(jnp.int32)
    ew = edge_attr.astype(jnp.float32)

    degp = _deg_call(dst, ew).reshape(NC, N_NODES)
    deg = degp[0] + degp[1] + 1.0
    dinv = jnp.where(deg > 0, lax.rsqrt(deg), 0.0)
    d2 = (dinv * dinv)[:, None]
    nrm = _norm_call(src, dst, ew, dinv)

    spmm128 = _make_spmm(128)
    spmm64 = _make_spmm(64)

    y1 = spmm128(x, src, dst, nrm)
    hidden = _dense(y1, x, d2, enc1_W, enc1_b, True)

    y2 = spmm128(hidden, src, dst, nrm)
    eps = jax.random.normal(jax.random.key(42), (N_NODES, mu_W.shape[1]),
                            jnp.float32)
    mu, logvar, z = _stageb(y2, hidden, d2, mu_W, mu_b, lv_W, lv_b, eps)

    y3 = spmm64(z, src, dst, nrm)
    dh = _dense(y3, z, d2, dec1_W, dec1_b, True)

    y4 = spmm128(dh, src, dst, nrm)
    reconstructed_x = _dense(y4, dh, d2, dec2_W, dec2_b, False)
    return (reconstructed_x, mu, logvar)


# speedup vs baseline: 12.3783x; 1.5408x over previous
import functools

import jax
import jax.numpy as jnp
from jax import lax
from jax.experimental import pallas as pl
from jax.experimental.pallas import tpu as pltpu
from jax.experimental.pallas import tpu_sc as plsc

N_NODES = 10000
N_EDGES = 320000
NC, NS, LANES = 2, 16, 16
NW = NC * NS
EPW = N_EDGES // NW
EC = 80
NCH = EPW // EC
RPT = N_NODES // NS


def _worker_ids():
    c = lax.axis_index("c")
    s = lax.axis_index("s")
    return c, s, s * NC + c


def _deg_body(dst_hbm, ew_hbm, out_hbm, dstb, ewb, zb, acc):
    c, s, w = _worker_ids()

    def zrow(i, _):
        zb[pl.ds(i * 16, 16)] = jnp.zeros((16,), jnp.float32)
        return 0
    lax.fori_loop(0, 40, zrow, 0)

    @pl.when(s < NS - 1)
    def _():
        pltpu.sync_copy(zb.at[pl.ds(0, 624)],
                        acc.at[pl.ds(pl.multiple_of(s * 624, 8), 624)])

    @pl.when(s == NS - 1)
    def _():
        pltpu.sync_copy(zb, acc.at[pl.ds((NS - 1) * 624, 640)])

    plsc.subcore_barrier()

    def chunk(i, _):
        off = pl.multiple_of(w * EPW + i * EC, 8)
        pltpu.sync_copy(dst_hbm.at[pl.ds(off, EC)], dstb)
        pltpu.sync_copy(ew_hbm.at[pl.ds(off, EC)], ewb)
        pltpu.sync_copy(ewb, acc.at[dstb], add=True)
        return 0
    lax.fori_loop(0, NCH, chunk, 0)

    plsc.subcore_barrier()

    @pl.when(s < NS - 1)
    def _():
        o = pl.multiple_of(s * 624, 8)
        pltpu.sync_copy(acc.at[pl.ds(o, 624)], zb.at[pl.ds(0, 624)])
        pltpu.sync_copy(zb.at[pl.ds(0, 624)],
                        out_hbm.at[pl.ds(c * N_NODES + o, 624)])

    @pl.when(s == NS - 1)
    def _():
        o = pl.multiple_of(c * N_NODES + (NS - 1) * 624, 8)
        pltpu.sync_copy(acc.at[pl.ds((NS - 1) * 624, 640)], zb)
        pltpu.sync_copy(zb, out_hbm.at[pl.ds(o, 640)])


_deg_call = pl.kernel(
    _deg_body,
    out_type=jax.ShapeDtypeStruct((NC * N_NODES,), jnp.float32),
    mesh=plsc.VectorSubcoreMesh(core_axis_name="c", subcore_axis_name="s"),
    compiler_params=pltpu.CompilerParams(use_tc_tiling_on_sc=False,
                                         needs_layout_passes=False),
    scratch_types=[
        pltpu.VMEM((EC,), jnp.int32),
        pltpu.VMEM((EC,), jnp.float32),
        pltpu.VMEM((640,), jnp.float32),
        pltpu.VMEM_SHARED((N_NODES,), jnp.float32),
    ],
)


NC2 = 400
NCH2 = EPW // NC2


def _norm_body(src_hbm, dst_hbm, ew_hbm, dinv_hbm, out_hbm, dv, srcb, dstb, ewb, nb):
    _, _, w = _worker_ids()
    pltpu.sync_copy(dinv_hbm, dv)

    def chunk(i, _):
        off = pl.multiple_of(w * EPW + i * NC2, 8)
        pltpu.sync_copy(src_hbm.at[pl.ds(off, NC2)], srcb)
        pltpu.sync_copy(dst_hbm.at[pl.ds(off, NC2)], dstb)
        pltpu.sync_copy(ew_hbm.at[pl.ds(off, NC2)], ewb)

        def inner(j, _):
            sl = pl.ds(j * 16, 16)
            nv = (plsc.load_gather(dv, [srcb[sl]]) * ewb[sl]
                  * plsc.load_gather(dv, [dstb[sl]]))
            nb[sl] = nv
            return 0
        lax.fori_loop(0, NC2 // 16, inner, 0)
        pltpu.sync_copy(nb, out_hbm.at[pl.ds(off, NC2)])
        return 0
    lax.fori_loop(0, NCH2, chunk, 0)


_norm_call = pl.kernel(
    _norm_body,
    out_type=jax.ShapeDtypeStruct((N_EDGES,), jnp.float32),
    mesh=plsc.VectorSubcoreMesh(core_axis_name="c", subcore_axis_name="s"),
    compiler_params=pltpu.CompilerParams(use_tc_tiling_on_sc=False,
                                         needs_layout_passes=False),
    scratch_types=[
        pltpu.VMEM((N_NODES,), jnp.float32),
        pltpu.VMEM((NC2,), jnp.int32),
        pltpu.VMEM((NC2,), jnp.int32),
        pltpu.VMEM((NC2,), jnp.float32),
        pltpu.VMEM((NC2,), jnp.float32),
    ],
)


def _spmm_body(F, x_hbm, ei_hbm, nrm_hbm, out_hbm,
               eib0, eib1, nb0, nb1, rows0, rows1, sem0, sem1, zb, acc):
    c, s, w = _worker_ids()
    bufs = ((eib0, nb0, rows0, sem0), (eib1, nb1, rows1, sem1))

    def zrow(i, _):
        for j in range(F // 16):
            zb[i, pl.ds(j * 16, 16)] = jnp.zeros((16,), jnp.float32)
        return 0
    lax.fori_loop(0, 104, zrow, 0)
    for k in range(6):
        pltpu.sync_copy(zb, acc.at[pl.ds(s * 624 + k * 104, 104)])

    @pl.when(s == NS - 1)
    def _():
        pltpu.sync_copy(zb.at[pl.ds(0, 16)], acc.at[pl.ds(N_NODES - 16, 16)])

    plsc.subcore_barrier()

    def _prefetch(i, b):
        eib, nb, rows, sem = bufs[b]
        off = pl.multiple_of(w * EPW + i * EC, 8)
        pltpu.sync_copy(ei_hbm.at[:, pl.ds(off, EC)], eib)
        pltpu.sync_copy(nrm_hbm.at[pl.ds(off, EC)], nb)
        pltpu.async_copy(x_hbm.at[eib.at[0]], rows, sem)

    def _finish(b):
        eib, nb, rows, sem = bufs[b]
        pltpu.make_async_copy(x_hbm.at[eib.at[0]], rows, sem).wait()

    UNR = 4

    def _scale_scatter(b):
        eib, nb, rows, sem = bufs[b]

        def scale(g, _):
            for u in range(UNR):
                r = g * UNR + u
                nbc = plsc.load_gather(nb, [jnp.full((LANES,), r, jnp.int32)])
                for j in range(F // 16):
                    rows[r, pl.ds(j * 16, 16)] = rows[r, pl.ds(j * 16, 16)] * nbc
            return 0
        lax.fori_loop(0, EC // UNR, scale, 0)
        pltpu.sync_copy(rows, acc.at[eib.at[1]], add=True)

    _prefetch(0, 0)

    def pair(k, _):
        i0 = 2 * k
        for b in range(2):
            _finish(b)
            _prefetch(i0 + b + 1, 1 - b)
            _scale_scatter(b)
        return 0
    lax.fori_loop(0, (NCH - 1) // 2, pair, 0)
    _finish(0)
    _scale_scatter(0)

    plsc.subcore_barrier()

    for k in range(6):
        o = pl.multiple_of(s * 624 + k * 104, 8)
        pltpu.sync_copy(acc.at[pl.ds(o, 104)], zb)
        pltpu.sync_copy(zb, out_hbm.at[c, pl.ds(o, 104)])

    @pl.when(s == NS - 1)
    def _():
        pltpu.sync_copy(acc.at[pl.ds(N_NODES - 16, 16)], zb.at[pl.ds(0, 16)])
        pltpu.sync_copy(zb.at[pl.ds(0, 16)],
                        out_hbm.at[c, pl.ds(N_NODES - 16, 16)])


@functools.cache
def _make_spmm(F):
    return pl.kernel(
        functools.partial(_spmm_body, F),
        out_type=jax.ShapeDtypeStruct((NC, N_NODES, F), jnp.float32),
        mesh=plsc.VectorSubcoreMesh(core_axis_name="c", subcore_axis_name="s"),
        compiler_params=pltpu.CompilerParams(use_tc_tiling_on_sc=False,
                                             needs_layout_passes=False),
        scratch_types=[
            pltpu.VMEM((2, EC), jnp.int32),
            pltpu.VMEM((2, EC), jnp.int32),
            pltpu.VMEM((EC,), jnp.float32),
            pltpu.VMEM((EC,), jnp.float32),
            pltpu.VMEM((EC, F), jnp.float32),
            pltpu.VMEM((EC, F), jnp.float32),
            pltpu.SemaphoreType.DMA,
            pltpu.SemaphoreType.DMA,
            pltpu.VMEM((104, F), jnp.float32),
            pltpu.VMEM_SHARED((N_NODES, F), jnp.float32),
        ],
    )


BS = 1000


def _dense_body(relu, y_ref, x_ref, d2_ref, w_ref, b_ref, o_ref):
    h = y_ref[0] + y_ref[1] + d2_ref[...] * x_ref[...]
    o = lax.dot_general(h, w_ref[...], (((1,), (0,)), ((), ())),
                        precision=lax.Precision.HIGHEST,
                        preferred_element_type=jnp.float32) + b_ref[...]
    o_ref[...] = jnp.maximum(o, 0.0) if relu else o


def _dense(y, x, d2, W, b, relu):
    n, fin = x.shape
    fout = W.shape[1]
    return pl.pallas_call(
        functools.partial(_dense_body, relu),
        grid=(n // BS,),
        in_specs=[
            pl.BlockSpec((NC, BS, fin), lambda i: (0, i, 0)),
            pl.BlockSpec((BS, fin), lambda i: (i, 0)),
            pl.BlockSpec((BS, 1), lambda i: (i, 0)),
            pl.BlockSpec((fin, fout), lambda i: (0, 0)),
            pl.BlockSpec((1, fout), lambda i: (0, 0)),
        ],
        out_specs=pl.BlockSpec((BS, fout), lambda i: (i, 0)),
        out_shape=jax.ShapeDtypeStruct((n, fout), jnp.float32),
    )(y, x, d2, W, b.reshape(1, -1))


def _stageb_body(y_ref, h_ref, d2_ref, wmu_ref, bmu_ref, wlv_ref, blv_ref,
                 eps_ref, mu_ref, lv_ref, z_ref):
    H = y_ref[0] + y_ref[1] + d2_ref[...] * h_ref[...]
    dn = (((1,), (0,)), ((), ()))
    mu = lax.dot_general(H, wmu_ref[...], dn, precision=lax.Precision.HIGHEST,
                         preferred_element_type=jnp.float32) + bmu_ref[...]
    lv = lax.dot_general(H, wlv_ref[...], dn, precision=lax.Precision.HIGHEST,
                         preferred_element_type=jnp.float32) + blv_ref[...]
    mu_ref[...] = mu
    lv_ref[...] = lv
    z_ref[...] = mu + eps_ref[...] * jnp.exp(0.5 * lv)


def _stageb(y, hidden, d2, mu_W, mu_b, lv_W, lv_b, eps):
    n, fin = hidden.shape
    fout = mu_W.shape[1]
    return pl.pallas_call(
        _stageb_body,
        grid=(n // BS,),
        in_specs=[
            pl.BlockSpec((NC, BS, fin), lambda i: (0, i, 0)),
            pl.BlockSpec((BS, fin), lambda i: (i, 0)),
            pl.BlockSpec((BS, 1), lambda i: (i, 0)),
            pl.BlockSpec((fin, fout), lambda i: (0, 0)),
            pl.BlockSpec((1, fout), lambda i: (0, 0)),
            pl.BlockSpec((fin, fout), lambda i: (0, 0)),
            pl.BlockSpec((1, fout), lambda i: (0, 0)),
            pl.BlockSpec((BS, fout), lambda i: (i, 0)),
        ],
        out_specs=[
            pl.BlockSpec((BS, fout), lambda i: (i, 0)),
            pl.BlockSpec((BS, fout), lambda i: (i, 0)),
            pl.BlockSpec((BS, fout), lambda i: (i, 0)),
        ],
        out_shape=[
            jax.ShapeDtypeStruct((n, fout), jnp.float32),
            jax.ShapeDtypeStruct((n, fout), jnp.float32),
            jax.ShapeDtypeStruct((n, fout), jnp.float32),
        ],
    )(y, hidden, d2, mu_W, mu_b.reshape(1, -1), lv_W, lv_b.reshape(1, -1), eps)


def kernel(x, edge_index, edge_attr, enc1_W, enc1_b, mu_W, mu_b, lv_W, lv_b,
           dec1_W, dec1_b, dec2_W, dec2_b):
    ei = edge_index.astype(jnp.int32)
    src = ei[0]
    dst = ei[1]
    ew = edge_attr.astype(jnp.float32)

    degp = _deg_call(dst, ew).reshape(NC, N_NODES)
    deg = degp[0] + degp[1] + 1.0
    dinv = jnp.where(deg > 0, lax.rsqrt(deg), 0.0)
    d2 = (dinv * dinv)[:, None]
    nrm = _norm_call(src, dst, ew, dinv)

    spmm128 = _make_spmm(128)
    spmm64 = _make_spmm(64)

    y1 = spmm128(x, ei, nrm)
    hidden = _dense(y1, x, d2, enc1_W, enc1_b, True)

    y2 = spmm128(hidden, ei, nrm)
    eps = jax.random.normal(jax.random.key(42), (N_NODES, mu_W.shape[1]),
                            jnp.float32)
    mu, logvar, z = _stageb(y2, hidden, d2, mu_W, mu_b, lv_W, lv_b, eps)

    y3 = spmm64(z, ei, nrm)
    dh = _dense(y3, z, d2, dec1_W, dec1_b, True)

    y4 = spmm128(dh, ei, nrm)
    reconstructed_x = _dense(y4, dh, d2, dec2_W, dec2_b, False)
    return (reconstructed_x, mu, logvar)


# speedup vs baseline: 15.0281x; 1.2141x over previous
import functools

import jax
import jax.numpy as jnp
from jax import lax
from jax.experimental import pallas as pl
from jax.experimental.pallas import tpu as pltpu
from jax.experimental.pallas import tpu_sc as plsc

N_NODES = 10000
N_EDGES = 320000
NC, NS, LANES = 2, 16, 16
NW = NC * NS
EPW = N_EDGES // NW
EC = 128
NCHF = EPW // EC
UNR = 8


def _worker_ids():
    c = lax.axis_index("c")
    s = lax.axis_index("s")
    return c, s, s * NC + c


def _deg_body(dst_hbm, ew_hbm, out_hbm, dstb, ewb, zb, dsem, acc):
    c, s, w = _worker_ids()

    def zrow(i, _):
        zb[pl.ds(i * 16, 16)] = jnp.zeros((16,), jnp.float32)
        return 0
    lax.fori_loop(0, 40, zrow, 0)

    @pl.when(s < NS - 1)
    def _():
        pltpu.sync_copy(zb.at[pl.ds(0, 624)],
                        acc.at[pl.ds(pl.multiple_of(s * 624, 8), 624)])

    @pl.when(s == NS - 1)
    def _():
        pltpu.sync_copy(zb, acc.at[pl.ds((NS - 1) * 624, 640)])

    plsc.subcore_barrier()

    def chunk(i, _):
        r0 = w * (EPW // 80) + i * 5
        pltpu.sync_copy(dst_hbm.at[pl.ds(r0, 5)], dstb)
        pltpu.sync_copy(ew_hbm.at[pl.ds(r0, 5)], ewb)
        for j in range(5):
            pltpu.async_copy(ewb.at[j], acc.at[dstb.at[j]], dsem, add=True)
        for j in range(5):
            pltpu.make_async_copy(ewb.at[0], acc.at[dstb.at[0]], dsem).wait()
        return 0
    lax.fori_loop(0, 25, chunk, 0)

    plsc.subcore_barrier()

    @pl.when(s < NS - 1)
    def _():
        o = pl.multiple_of(s * 624, 8)
        pltpu.sync_copy(acc.at[pl.ds(o, 624)], zb.at[pl.ds(0, 624)])
        pltpu.sync_copy(zb.at[pl.ds(0, 624)],
                        out_hbm.at[pl.ds(c * N_NODES + o, 624)])

    @pl.when(s == NS - 1)
    def _():
        o = pl.multiple_of(c * N_NODES + (NS - 1) * 624, 8)
        pltpu.sync_copy(acc.at[pl.ds((NS - 1) * 624, 640)], zb)
        pltpu.sync_copy(zb, out_hbm.at[pl.ds(o, 640)])


_deg_call = pl.kernel(
    _deg_body,
    out_type=jax.ShapeDtypeStruct((NC * N_NODES,), jnp.float32),
    mesh=plsc.VectorSubcoreMesh(core_axis_name="c", subcore_axis_name="s"),
    compiler_params=pltpu.CompilerParams(use_tc_tiling_on_sc=False,
                                         needs_layout_passes=False),
    scratch_types=[
        pltpu.VMEM((5, 80), jnp.int32),
        pltpu.VMEM((5, 80), jnp.float32),
        pltpu.VMEM((640,), jnp.float32),
        pltpu.SemaphoreType.DMA,
        pltpu.VMEM_SHARED((N_NODES,), jnp.float32),
    ],
)


NC2 = 400
NCH2 = EPW // NC2


def _norm_body(src_hbm, dst_hbm, ew_hbm, dinv_hbm, out_hbm, dv, srcb, dstb, ewb, nb):
    _, _, w = _worker_ids()
    pltpu.sync_copy(dinv_hbm, dv)

    def chunk(i, _):
        off = pl.multiple_of(w * EPW + i * NC2, 8)
        pltpu.sync_copy(src_hbm.at[pl.ds(off, NC2)], srcb)
        pltpu.sync_copy(dst_hbm.at[pl.ds(off, NC2)], dstb)
        pltpu.sync_copy(ew_hbm.at[pl.ds(off, NC2)], ewb)

        def inner(j, _):
            sl = pl.ds(j * 16, 16)
            nv = (plsc.load_gather(dv, [srcb[sl]]) * ewb[sl]
                  * plsc.load_gather(dv, [dstb[sl]]))
            nb[sl] = nv
            return 0
        lax.fori_loop(0, NC2 // 16, inner, 0)
        pltpu.sync_copy(nb, out_hbm.at[pl.ds(off, NC2)])
        return 0
    lax.fori_loop(0, NCH2, chunk, 0)


_norm_call = pl.kernel(
    _norm_body,
    out_type=jax.ShapeDtypeStruct((N_EDGES,), jnp.float32),
    mesh=plsc.VectorSubcoreMesh(core_axis_name="c", subcore_axis_name="s"),
    compiler_params=pltpu.CompilerParams(use_tc_tiling_on_sc=False,
                                         needs_layout_passes=False),
    scratch_types=[
        pltpu.VMEM((N_NODES,), jnp.float32),
        pltpu.VMEM((NC2,), jnp.int32),
        pltpu.VMEM((NC2,), jnp.int32),
        pltpu.VMEM((NC2,), jnp.float32),
        pltpu.VMEM((NC2,), jnp.float32),
    ],
)


def _spmm_body(F, x_hbm, ei_hbm, nrm_hbm, out_hbm,
               eib0, eib1, nb0, nb1, rows0, rows1, gsem0, gsem1,
               ssem0, ssem1, zb, acc):
    c, s, w = _worker_ids()
    bufs = ((eib0, nb0, rows0, gsem0, ssem0), (eib1, nb1, rows1, gsem1, ssem1))

    def zrow(i, _):
        for j in range(F // 16):
            zb[i, pl.ds(j * 16, 16)] = jnp.zeros((16,), jnp.float32)
        return 0
    lax.fori_loop(0, 104, zrow, 0)
    for k in range(6):
        pltpu.sync_copy(zb, acc.at[pl.ds(s * 624 + k * 104, 104)])

    @pl.when(s == NS - 1)
    def _():
        pltpu.sync_copy(zb.at[pl.ds(0, 16)], acc.at[pl.ds(N_NODES - 16, 16)])

    plsc.subcore_barrier()

    def _prefetch(i, b):
        eib, nb, rows, gsem, ssem = bufs[b]
        off = pl.multiple_of(w * EPW + i * EC, 8)
        pltpu.sync_copy(ei_hbm.at[:, pl.ds(off, EC)], eib)
        pltpu.sync_copy(nrm_hbm.at[pl.ds(off, EC)], nb)
        pltpu.async_copy(x_hbm.at[eib.at[0]], rows, gsem)

    def _finish(b):
        eib, nb, rows, gsem, ssem = bufs[b]
        pltpu.make_async_copy(x_hbm.at[eib.at[0]], rows, gsem).wait()

    def _scale(b, nrows):
        eib, nb, rows, gsem, ssem = bufs[b]

        def srow(g, _):
            for u in range(UNR):
                r = g * UNR + u
                nbc = plsc.load_gather(nb, [jnp.full((LANES,), r, jnp.int32)])
                for j in range(F // 16):
                    rows[r, pl.ds(j * 16, 16)] = rows[r, pl.ds(j * 16, 16)] * nbc
            return 0
        lax.fori_loop(0, nrows // UNR, srow, 0)

    def _scatter(b):
        eib, nb, rows, gsem, ssem = bufs[b]
        pltpu.async_copy(rows, acc.at[eib.at[1]], ssem, add=True)

    def _wait_scatter(b):
        eib, nb, rows, gsem, ssem = bufs[b]
        pltpu.make_async_copy(rows, acc.at[eib.at[1]], ssem).wait()

    _prefetch(0, 0)

    def pair(k, _):
        for b in range(2):
            i = 2 * k + b

            @pl.when(i >= 1)
            def _():
                _wait_scatter(1 - b)
            _finish(b)

            @pl.when(i < NCHF - 1)
            def _():
                _prefetch(i + 1, 1 - b)
            _scale(b, EC)
            _scatter(b)
        return 0
    lax.fori_loop(0, NCHF // 2, pair, 0)
    _wait_scatter(1)

    offt = pl.multiple_of(w * EPW + NCHF * EC, 8)
    eib, nb, rows, gsem, ssem = bufs[0]
    pltpu.sync_copy(ei_hbm.at[:, pl.ds(offt, 16)], eib.at[:, pl.ds(0, 16)])
    pltpu.sync_copy(nrm_hbm.at[pl.ds(offt, 16)], nb.at[pl.ds(0, 16)])
    pltpu.async_copy(x_hbm.at[eib.at[0, pl.ds(0, 16)]],
                     rows.at[pl.ds(0, 16)], gsem)
    pltpu.make_async_copy(x_hbm.at[eib.at[0, pl.ds(0, 16)]],
                          rows.at[pl.ds(0, 16)], gsem).wait()

    def trow(r, _):
        nbc = plsc.load_gather(nb, [jnp.full((LANES,), r, jnp.int32)])
        for j in range(F // 16):
            rows[r, pl.ds(j * 16, 16)] = rows[r, pl.ds(j * 16, 16)] * nbc
        return 0
    lax.fori_loop(0, 16, trow, 0)
    pltpu.sync_copy(rows.at[pl.ds(0, 16)],
                    acc.at[eib.at[1, pl.ds(0, 16)]], add=True)

    plsc.subcore_barrier()

    for k in range(6):
        o = pl.multiple_of(s * 624 + k * 104, 8)
        pltpu.sync_copy(acc.at[pl.ds(o, 104)], zb)
        pltpu.sync_copy(zb, out_hbm.at[c, pl.ds(o, 104)])

    @pl.when(s == NS - 1)
    def _():
        pltpu.sync_copy(acc.at[pl.ds(N_NODES - 16, 16)], zb.at[pl.ds(0, 16)])
        pltpu.sync_copy(zb.at[pl.ds(0, 16)],
                        out_hbm.at[c, pl.ds(N_NODES - 16, 16)])


@functools.cache
def _make_spmm(F):
    return pl.kernel(
        functools.partial(_spmm_body, F),
        out_type=jax.ShapeDtypeStruct((NC, N_NODES, F), jnp.float32),
        mesh=plsc.VectorSubcoreMesh(core_axis_name="c", subcore_axis_name="s"),
        compiler_params=pltpu.CompilerParams(use_tc_tiling_on_sc=False,
                                             needs_layout_passes=False),
        scratch_types=[
            pltpu.VMEM((2, EC), jnp.int32),
            pltpu.VMEM((2, EC), jnp.int32),
            pltpu.VMEM((EC,), jnp.float32),
            pltpu.VMEM((EC,), jnp.float32),
            pltpu.VMEM((EC, F), jnp.float32),
            pltpu.VMEM((EC, F), jnp.float32),
            pltpu.SemaphoreType.DMA,
            pltpu.SemaphoreType.DMA,
            pltpu.SemaphoreType.DMA,
            pltpu.SemaphoreType.DMA,
            pltpu.VMEM((104, F), jnp.float32),
            pltpu.VMEM_SHARED((N_NODES, F), jnp.float32),
        ],
    )


BS = 1000


def _dense_body(relu, y_ref, x_ref, d2_ref, w_ref, b_ref, o_ref):
    h = y_ref[0] + y_ref[1] + d2_ref[...] * x_ref[...]
    o = lax.dot_general(h, w_ref[...], (((1,), (0,)), ((), ())),
                        precision=lax.Precision.HIGHEST,
                        preferred_element_type=jnp.float32) + b_ref[...]
    o_ref[...] = jnp.maximum(o, 0.0) if relu else o


def _dense(y, x, d2, W, b, relu):
    n, fin = x.shape
    fout = W.shape[1]
    return pl.pallas_call(
        functools.partial(_dense_body, relu),
        grid=(n // BS,),
        in_specs=[
            pl.BlockSpec((NC, BS, fin), lambda i: (0, i, 0)),
            pl.BlockSpec((BS, fin), lambda i: (i, 0)),
            pl.BlockSpec((BS, 1), lambda i: (i, 0)),
            pl.BlockSpec((fin, fout), lambda i: (0, 0)),
            pl.BlockSpec((1, fout), lambda i: (0, 0)),
        ],
        out_specs=pl.BlockSpec((BS, fout), lambda i: (i, 0)),
        out_shape=jax.ShapeDtypeStruct((n, fout), jnp.float32),
    )(y, x, d2, W, b.reshape(1, -1))


def _stageb_body(y_ref, h_ref, d2_ref, wmu_ref, bmu_ref, wlv_ref, blv_ref,
                 eps_ref, mu_ref, lv_ref, z_ref):
    H = y_ref[0] + y_ref[1] + d2_ref[...] * h_ref[...]
    dn = (((1,), (0,)), ((), ()))
    mu = lax.dot_general(H, wmu_ref[...], dn, precision=lax.Precision.HIGHEST,
                         preferred_element_type=jnp.float32) + bmu_ref[...]
    lv = lax.dot_general(H, wlv_ref[...], dn, precision=lax.Precision.HIGHEST,
                         preferred_element_type=jnp.float32) + blv_ref[...]
    mu_ref[...] = mu
    lv_ref[...] = lv
    z_ref[...] = mu + eps_ref[...] * jnp.exp(0.5 * lv)


def _stageb(y, hidden, d2, mu_W, mu_b, lv_W, lv_b, eps):
    n, fin = hidden.shape
    fout = mu_W.shape[1]
    return pl.pallas_call(
        _stageb_body,
        grid=(n // BS,),
        in_specs=[
            pl.BlockSpec((NC, BS, fin), lambda i: (0, i, 0)),
            pl.BlockSpec((BS, fin), lambda i: (i, 0)),
            pl.BlockSpec((BS, 1), lambda i: (i, 0)),
            pl.BlockSpec((fin, fout), lambda i: (0, 0)),
            pl.BlockSpec((1, fout), lambda i: (0, 0)),
            pl.BlockSpec((fin, fout), lambda i: (0, 0)),
            pl.BlockSpec((1, fout), lambda i: (0, 0)),
            pl.BlockSpec((BS, fout), lambda i: (i, 0)),
        ],
        out_specs=[
            pl.BlockSpec((BS, fout), lambda i: (i, 0)),
            pl.BlockSpec((BS, fout), lambda i: (i, 0)),
            pl.BlockSpec((BS, fout), lambda i: (i, 0)),
        ],
        out_shape=[
            jax.ShapeDtypeStruct((n, fout), jnp.float32),
            jax.ShapeDtypeStruct((n, fout), jnp.float32),
            jax.ShapeDtypeStruct((n, fout), jnp.float32),
        ],
    )(y, hidden, d2, mu_W, mu_b.reshape(1, -1), lv_W, lv_b.reshape(1, -1), eps)


def kernel(x, edge_index, edge_attr, enc1_W, enc1_b, mu_W, mu_b, lv_W, lv_b,
           dec1_W, dec1_b, dec2_W, dec2_b):
    ei = edge_index.astype(jnp.int32)
    src = ei[0]
    dst = ei[1]
    ew = edge_attr.astype(jnp.float32)

    degp = _deg_call(dst.reshape(-1, 80),
                     ew.reshape(-1, 80)).reshape(NC, N_NODES)
    deg = degp[0] + degp[1] + 1.0
    dinv = jnp.where(deg > 0, lax.rsqrt(deg), 0.0)
    d2 = (dinv * dinv)[:, None]
    nrm = _norm_call(src, dst, ew, dinv)

    spmm128 = _make_spmm(128)
    spmm64 = _make_spmm(64)

    y1 = spmm128(x, ei, nrm)
    hidden = _dense(y1, x, d2, enc1_W, enc1_b, True)

    y2 = spmm128(hidden, ei, nrm)
    eps = jax.random.normal(jax.random.key(42), (N_NODES, mu_W.shape[1]),
                            jnp.float32)
    mu, logvar, z = _stageb(y2, hidden, d2, mu_W, mu_b, lv_W, lv_b, eps)

    y3 = spmm64(z, ei, nrm)
    dh = _dense(y3, z, d2, dec1_W, dec1_b, True)

    y4 = spmm128(dh, ei, nrm)
    reconstructed_x = _dense(y4, dh, d2, dec2_W, dec2_b, False)
    return (reconstructed_x, mu, logvar)


# speedup vs baseline: 20.7227x; 1.3789x over previous
import functools

import jax
import jax.numpy as jnp
from jax import lax
from jax.experimental import pallas as pl
from jax.experimental.pallas import tpu as pltpu
from jax.experimental.pallas import tpu_sc as plsc

N_NODES = 10000
N_EDGES = 320000
NC, NS, LANES = 2, 16, 16
NW = NC * NS
EPW = N_EDGES // NW
EC = 104
NCHF = EPW // EC
UNR = 8


def _worker_ids():
    c = lax.axis_index("c")
    s = lax.axis_index("s")
    return c, s, s * NC + c


def _deg_body(dst_hbm, ew_hbm, out_hbm, dstb, ewb, zb, dsem, acc):
    c, s, w = _worker_ids()

    def zrow(i, _):
        zb[pl.ds(i * 16, 16)] = jnp.zeros((16,), jnp.float32)
        return 0
    lax.fori_loop(0, 40, zrow, 0)

    @pl.when(s < NS - 1)
    def _():
        pltpu.sync_copy(zb.at[pl.ds(0, 624)],
                        acc.at[pl.ds(pl.multiple_of(s * 624, 8), 624)])

    @pl.when(s == NS - 1)
    def _():
        pltpu.sync_copy(zb, acc.at[pl.ds((NS - 1) * 624, 640)])

    plsc.subcore_barrier()

    def chunk(i, _):
        r0 = w * (EPW // 80) + i * 5
        pltpu.sync_copy(dst_hbm.at[pl.ds(r0, 5)], dstb)
        pltpu.sync_copy(ew_hbm.at[pl.ds(r0, 5)], ewb)
        for j in range(5):
            pltpu.async_copy(ewb.at[j], acc.at[dstb.at[j]], dsem, add=True)
        for j in range(5):
            pltpu.make_async_copy(ewb.at[0], acc.at[dstb.at[0]], dsem).wait()
        return 0
    lax.fori_loop(0, 25, chunk, 0)

    plsc.subcore_barrier()

    @pl.when(s < NS - 1)
    def _():
        o = pl.multiple_of(s * 624, 8)
        pltpu.sync_copy(acc.at[pl.ds(o, 624)], zb.at[pl.ds(0, 624)])
        pltpu.sync_copy(zb.at[pl.ds(0, 624)],
                        out_hbm.at[pl.ds(c * N_NODES + o, 624)])

    @pl.when(s == NS - 1)
    def _():
        o = pl.multiple_of(c * N_NODES + (NS - 1) * 624, 8)
        pltpu.sync_copy(acc.at[pl.ds((NS - 1) * 624, 640)], zb)
        pltpu.sync_copy(zb, out_hbm.at[pl.ds(o, 640)])


_deg_call = pl.kernel(
    _deg_body,
    out_type=jax.ShapeDtypeStruct((NC * N_NODES,), jnp.float32),
    mesh=plsc.VectorSubcoreMesh(core_axis_name="c", subcore_axis_name="s"),
    compiler_params=pltpu.CompilerParams(use_tc_tiling_on_sc=False,
                                         needs_layout_passes=False),
    scratch_types=[
        pltpu.VMEM((5, 80), jnp.int32),
        pltpu.VMEM((5, 80), jnp.float32),
        pltpu.VMEM((640,), jnp.float32),
        pltpu.SemaphoreType.DMA,
        pltpu.VMEM_SHARED((N_NODES,), jnp.float32),
    ],
)


NC2 = 400
NCH2 = EPW // NC2


def _norm_body(src_hbm, dst_hbm, ew_hbm, dinv_hbm, out_hbm, dv, srcb, dstb, ewb, nbi):
    _, _, w = _worker_ids()
    pltpu.sync_copy(dinv_hbm, dv)

    def chunk(i, _):
        off = pl.multiple_of(w * EPW + i * NC2, 8)
        pltpu.sync_copy(src_hbm.at[pl.ds(off, NC2)], srcb)
        pltpu.sync_copy(dst_hbm.at[pl.ds(off, NC2)], dstb)
        pltpu.sync_copy(ew_hbm.at[pl.ds(off, NC2)], ewb)

        def inner(j, _):
            sl = pl.ds(j * 16, 16)
            nv = (plsc.load_gather(dv, [srcb[sl]]) * ewb[sl]
                  * plsc.load_gather(dv, [dstb[sl]]))
            nbi[sl] = plsc.bitcast(nv, jnp.int32)
            return 0
        lax.fori_loop(0, NC2 // 16, inner, 0)
        pltpu.sync_copy(srcb, out_hbm.at[0, pl.ds(off, NC2)])
        pltpu.sync_copy(dstb, out_hbm.at[1, pl.ds(off, NC2)])
        pltpu.sync_copy(nbi, out_hbm.at[2, pl.ds(off, NC2)])
        return 0
    lax.fori_loop(0, NCH2, chunk, 0)


_norm_call = pl.kernel(
    _norm_body,
    out_type=jax.ShapeDtypeStruct((3, N_EDGES), jnp.int32),
    mesh=plsc.VectorSubcoreMesh(core_axis_name="c", subcore_axis_name="s"),
    compiler_params=pltpu.CompilerParams(use_tc_tiling_on_sc=False,
                                         needs_layout_passes=False),
    scratch_types=[
        pltpu.VMEM((N_NODES,), jnp.float32),
        pltpu.VMEM((NC2,), jnp.int32),
        pltpu.VMEM((NC2,), jnp.int32),
        pltpu.VMEM((NC2,), jnp.float32),
        pltpu.VMEM((NC2,), jnp.int32),
    ],
)


def _spmm_body(F, x_hbm, pk_hbm, out_hbm,
               pk0, pk1, pk2, rows0, rows1, rows2, isem0, isem1, isem2,
               gsem0, gsem1, gsem2, ssem0, ssem1, ssem2, acc):
    c, s, w = _worker_ids()
    bufs = ((pk0, rows0, isem0, gsem0, ssem0),
            (pk1, rows1, isem1, gsem1, ssem1),
            (pk2, rows2, isem2, gsem2, ssem2))

    def zrow(i, _):
        for j in range(F // 16):
            rows0[i, pl.ds(j * 16, 16)] = jnp.zeros((16,), jnp.float32)
        return 0
    lax.fori_loop(0, EC, zrow, 0)
    for k in range(6):
        pltpu.sync_copy(rows0, acc.at[pl.ds(s * 624 + k * 104, 104)])

    @pl.when(s == NS - 1)
    def _():
        pltpu.sync_copy(rows0.at[pl.ds(0, 16)], acc.at[pl.ds(N_NODES - 16, 16)])

    plsc.subcore_barrier()

    def _idx_fetch(i, b):
        pk, rows, isem, gsem, ssem = bufs[b]
        off = pl.multiple_of(w * EPW + i * EC, 8)
        pltpu.async_copy(pk_hbm.at[:, pl.ds(off, EC)], pk, isem)

    def _idx_wait(b):
        pk, rows, isem, gsem, ssem = bufs[b]
        pltpu.make_async_copy(pk_hbm.at[:, pl.ds(0, EC)], pk, isem).wait()

    def _gather(b):
        pk, rows, isem, gsem, ssem = bufs[b]
        pltpu.async_copy(x_hbm.at[pk.at[0]], rows, gsem)

    def _gather_wait(b):
        pk, rows, isem, gsem, ssem = bufs[b]
        pltpu.make_async_copy(x_hbm.at[pk.at[0]], rows, gsem).wait()

    def _scale(b, nrows):
        pk, rows, isem, gsem, ssem = bufs[b]

        def srow(g, _):
            for u in range(UNR):
                r = g * UNR + u
                nbc = plsc.bitcast(
                    plsc.load_gather(pk.at[2], [jnp.full((LANES,), r, jnp.int32)]),
                    jnp.float32)
                for j in range(F // 16):
                    rows[r, pl.ds(j * 16, 16)] = rows[r, pl.ds(j * 16, 16)] * nbc
            return 0
        lax.fori_loop(0, nrows // UNR, srow, 0)

    def _scatter(b):
        pk, rows, isem, gsem, ssem = bufs[b]
        pltpu.async_copy(rows, acc.at[pk.at[1]], ssem, add=True)

    def _wait_scatter(b):
        pk, rows, isem, gsem, ssem = bufs[b]
        pltpu.make_async_copy(rows, acc.at[pk.at[1]], ssem).wait()

    _idx_fetch(0, 0)
    _idx_wait(0)
    _gather(0)
    _idx_fetch(1, 1)

    def triple(k, _):
        for b in range(3):
            i = 3 * k + b
            _gather_wait(b)
            nxt = (b + 1) % 3
            pre = (b + 2) % 3

            @pl.when(i + 1 < NCHF)
            def _():
                _idx_wait(nxt)
                _gather(nxt)

            @pl.when(i >= 1)
            def _():
                _wait_scatter(pre)

            @pl.when(i + 2 < NCHF)
            def _():
                _idx_fetch(i + 2, pre)
            _scale(b, EC)
            _scatter(b)
        return 0
    lax.fori_loop(0, NCHF // 3, triple, 0)
    _wait_scatter((NCHF - 1) % 3)

    offt = pl.multiple_of(w * EPW + NCHF * EC, 8)
    pk, rows, isem, gsem, ssem = bufs[0]
    pltpu.sync_copy(pk_hbm.at[:, pl.ds(offt, 16)], pk.at[:, pl.ds(0, 16)])
    pltpu.async_copy(x_hbm.at[pk.at[0, pl.ds(0, 16)]],
                     rows.at[pl.ds(0, 16)], gsem)
    pltpu.make_async_copy(x_hbm.at[pk.at[0, pl.ds(0, 16)]],
                          rows.at[pl.ds(0, 16)], gsem).wait()

    def trow(r, _):
        nbc = plsc.bitcast(
            plsc.load_gather(pk.at[2], [jnp.full((LANES,), r, jnp.int32)]),
            jnp.float32)
        for j in range(F // 16):
            rows[r, pl.ds(j * 16, 16)] = rows[r, pl.ds(j * 16, 16)] * nbc
        return 0
    lax.fori_loop(0, 16, trow, 0)
    pltpu.sync_copy(rows.at[pl.ds(0, 16)],
                    acc.at[pk.at[1, pl.ds(0, 16)]], add=True)

    plsc.subcore_barrier()

    for k in range(6):
        o = pl.multiple_of(s * 624 + k * 104, 8)
        stg, wsem = bufs[k % 3][1], bufs[k % 3][3]
        if k >= 3:
            pltpu.make_async_copy(stg, out_hbm.at[c, pl.ds(0, 104)], wsem).wait()
        pltpu.sync_copy(acc.at[pl.ds(o, 104)], stg)
        pltpu.async_copy(stg, out_hbm.at[c, pl.ds(o, 104)], wsem)
    for k in range(3):
        stg, wsem = bufs[k][1], bufs[k][3]
        pltpu.make_async_copy(stg, out_hbm.at[c, pl.ds(0, 104)], wsem).wait()

    @pl.when(s == NS - 1)
    def _():
        pltpu.sync_copy(acc.at[pl.ds(N_NODES - 16, 16)], rows0.at[pl.ds(0, 16)])
        pltpu.sync_copy(rows0.at[pl.ds(0, 16)],
                        out_hbm.at[c, pl.ds(N_NODES - 16, 16)])


@functools.cache
def _make_spmm(F):
    return pl.kernel(
        functools.partial(_spmm_body, F),
        out_type=jax.ShapeDtypeStruct((NC, N_NODES, F), jnp.float32),
        mesh=plsc.VectorSubcoreMesh(core_axis_name="c", subcore_axis_name="s"),
        compiler_params=pltpu.CompilerParams(use_tc_tiling_on_sc=False,
                                             needs_layout_passes=False),
        scratch_types=(
            [pltpu.VMEM((3, EC), jnp.int32)] * 3
            + [pltpu.VMEM((EC, F), jnp.float32)] * 3
            + [pltpu.SemaphoreType.DMA] * 9
            + [pltpu.VMEM_SHARED((N_NODES, F), jnp.float32)]
        ),
    )


BS = 1000


def _dense_body(relu, y_ref, x_ref, d2_ref, w_ref, b_ref, o_ref):
    h = y_ref[0] + y_ref[1] + d2_ref[...] * x_ref[...]
    o = lax.dot_general(h, w_ref[...], (((1,), (0,)), ((), ())),
                        precision=lax.Precision.HIGHEST,
                        preferred_element_type=jnp.float32) + b_ref[...]
    o_ref[...] = jnp.maximum(o, 0.0) if relu else o


def _dense(y, x, d2, W, b, relu):
    n, fin = x.shape
    fout = W.shape[1]
    return pl.pallas_call(
        functools.partial(_dense_body, relu),
        grid=(n // BS,),
        in_specs=[
            pl.BlockSpec((NC, BS, fin), lambda i: (0, i, 0)),
            pl.BlockSpec((BS, fin), lambda i: (i, 0)),
            pl.BlockSpec((BS, 1), lambda i: (i, 0)),
            pl.BlockSpec((fin, fout), lambda i: (0, 0)),
            pl.BlockSpec((1, fout), lambda i: (0, 0)),
        ],
        out_specs=pl.BlockSpec((BS, fout), lambda i: (i, 0)),
        out_shape=jax.ShapeDtypeStruct((n, fout), jnp.float32),
    )(y, x, d2, W, b.reshape(1, -1))


def _stageb_body(y_ref, h_ref, d2_ref, wmu_ref, bmu_ref, wlv_ref, blv_ref,
                 eps_ref, mu_ref, lv_ref, z_ref):
    H = y_ref[0] + y_ref[1] + d2_ref[...] * h_ref[...]
    dn = (((1,), (0,)), ((), ()))
    mu = lax.dot_general(H, wmu_ref[...], dn, precision=lax.Precision.HIGHEST,
                         preferred_element_type=jnp.float32) + bmu_ref[...]
    lv = lax.dot_general(H, wlv_ref[...], dn, precision=lax.Precision.HIGHEST,
                         preferred_element_type=jnp.float32) + blv_ref[...]
    mu_ref[...] = mu
    lv_ref[...] = lv
    z_ref[...] = mu + eps_ref[...] * jnp.exp(0.5 * lv)


def _stageb(y, hidden, d2, mu_W, mu_b, lv_W, lv_b, eps):
    n, fin = hidden.shape
    fout = mu_W.shape[1]
    return pl.pallas_call(
        _stageb_body,
        grid=(n // BS,),
        in_specs=[
            pl.BlockSpec((NC, BS, fin), lambda i: (0, i, 0)),
            pl.BlockSpec((BS, fin), lambda i: (i, 0)),
            pl.BlockSpec((BS, 1), lambda i: (i, 0)),
            pl.BlockSpec((fin, fout), lambda i: (0, 0)),
            pl.BlockSpec((1, fout), lambda i: (0, 0)),
            pl.BlockSpec((fin, fout), lambda i: (0, 0)),
            pl.BlockSpec((1, fout), lambda i: (0, 0)),
            pl.BlockSpec((BS, fout), lambda i: (i, 0)),
        ],
        out_specs=[
            pl.BlockSpec((BS, fout), lambda i: (i, 0)),
            pl.BlockSpec((BS, fout), lambda i: (i, 0)),
            pl.BlockSpec((BS, fout), lambda i: (i, 0)),
        ],
        out_shape=[
            jax.ShapeDtypeStruct((n, fout), jnp.float32),
            jax.ShapeDtypeStruct((n, fout), jnp.float32),
            jax.ShapeDtypeStruct((n, fout), jnp.float32),
        ],
    )(y, hidden, d2, mu_W, mu_b.reshape(1, -1), lv_W, lv_b.reshape(1, -1), eps)


def kernel(x, edge_index, edge_attr, enc1_W, enc1_b, mu_W, mu_b, lv_W, lv_b,
           dec1_W, dec1_b, dec2_W, dec2_b):
    ei = edge_index.astype(jnp.int32)
    src = ei[0]
    dst = ei[1]
    ew = edge_attr.astype(jnp.float32)

    degp = _deg_call(dst.reshape(-1, 80),
                     ew.reshape(-1, 80)).reshape(NC, N_NODES)
    deg = degp[0] + degp[1] + 1.0
    dinv = jnp.where(deg > 0, lax.rsqrt(deg), 0.0)
    d2 = (dinv * dinv)[:, None]
    pk = _norm_call(src, dst, ew, dinv)

    spmm128 = _make_spmm(128)
    spmm64 = _make_spmm(64)

    y1 = spmm128(x, pk)
    hidden = _dense(y1, x, d2, enc1_W, enc1_b, True)

    y2 = spmm128(hidden, pk)
    eps = jax.random.normal(jax.random.key(42), (N_NODES, mu_W.shape[1]),
                            jnp.float32)
    mu, logvar, z = _stageb(y2, hidden, d2, mu_W, mu_b, lv_W, lv_b, eps)

    y3 = spmm64(z, pk)
    dh = _dense(y3, z, d2, dec1_W, dec1_b, True)

    y4 = spmm128(dh, pk)
    reconstructed_x = _dense(y4, dh, d2, dec2_W, dec2_b, False)
    return (reconstructed_x, mu, logvar)
